# baseline (speedup 1.0000x reference)
import functools

import jax
import jax.numpy as jnp
from jax.experimental import pallas as pl
from jax.experimental.pallas import tpu as pltpu

_SQRT_2_OVER_PI = 0.7978845608028654
_GELU_C = 0.044715

_STRIDE = 3
_DIL = 2
_NPH = 3
_NTAP = 9
_UP = 48
_OWP = 32
_OHP = 32
_T_OF = (0, 2, 1)
_DT = (0, 1, 0)


def _gelu_tail(v1):
    v3 = v1 * v1
    v5 = (v3 * v1) * _GELU_C
    v7 = (v1 + v5) * _SQRT_2_OVER_PI
    v10 = (v1 * 0.5) * (1.0 + jnp.tanh(v7))
    return v10 + v5


def _conv_gelu_body(xs_ref, g_ref, b_ref, o_ref, *, lanes):
    acc = jnp.zeros((_OHP, 128), jnp.float32)
    for p in range(_NTAP):
        lhs = xs_ref[0, pl.ds(_DIL * p, _OHP), :]
        acc += jnp.dot(lhs, g_ref[p], preferred_element_type=jnp.float32)
    o_ref[0] = _gelu_tail(acc + b_ref[...])


@jax.jit
def kernel(x, w1, b1):
    N, Cin, H, W = map(int, x.shape)
    Cout, _, KH, KW = map(int, w1.shape)
    OH = (H - _DIL * (KH - 1) - 1) // _STRIDE + 1
    OW = (W - _DIL * (KW - 1) - 1) // _STRIDE + 1
    LB = Cin * W
    LANES = _NPH * LB

    xp = jnp.pad(x.astype(jnp.bfloat16),
                 ((0, 0), (0, 0), (0, _STRIDE * (_UP + 1) - H), (0, 0)))
    xr = xp.reshape(N, Cin, _UP + 1, _NPH, W)
    xst = jnp.stack(
        [xr[:, :, _DT[T]:_DT[T] + _UP, T, :] for T in range(_NPH)],
        axis=3)
    xs = xst.transpose(0, 2, 3, 1, 4).reshape(N, _UP, LANES)

    w8 = jnp.pad(w1.astype(jnp.bfloat16),
                 ((0, 0), (0, 0), (0, _NPH * _NTAP - KH), (0, 0)))
    w9 = w8.reshape(Cout, Cin, _NTAP, _NPH, KW)
    w10 = jnp.take(w9, jnp.array(_T_OF), axis=3)
    c_idx = jnp.arange(W)[:, None, None]
    w_idx = jnp.arange(_OWP)[None, :, None]
    k_idx = jnp.arange(KW)[None, None, :]
    sel = ((c_idx == _STRIDE * w_idx + _DIL * k_idx)
           & (w_idx < OW)).astype(jnp.bfloat16)
    g = jnp.einsum('oipTk,cwk->pTicow', w10, sel,
                   preferred_element_type=jnp.float32)
    g = g.reshape(_NTAP, LANES, Cout * _OWP).astype(jnp.bfloat16)

    bvec = jnp.repeat(b1.astype(jnp.float32), _OWP).reshape(1, Cout * _OWP)

    cost = pl.CostEstimate(
        flops=2 * N * _NTAP * _OHP * LANES * Cout * _OWP,
        transcendentals=N * _OHP * Cout * _OWP,
        bytes_accessed=xs.size * 2 + g.size * 2 + N * _OHP * Cout * _OWP * 4,
    )
    out2 = pl.pallas_call(
        functools.partial(_conv_gelu_body, lanes=LANES),
        out_shape=jax.ShapeDtypeStruct((N, _OHP, Cout * _OWP), jnp.float32),
        grid=(N,),
        in_specs=[
            pl.BlockSpec((1, _UP, LANES), lambda n: (n, 0, 0)),
            pl.BlockSpec((_NTAP, LANES, Cout * _OWP), lambda n: (0, 0, 0)),
            pl.BlockSpec((1, Cout * _OWP), lambda n: (0, 0)),
        ],
        out_specs=pl.BlockSpec((1, _OHP, Cout * _OWP), lambda n: (n, 0, 0)),
        compiler_params=pltpu.CompilerParams(
            dimension_semantics=("parallel",),
            vmem_limit_bytes=48 * 1024 * 1024),
        cost_estimate=cost,
    )(xs, g, bvec)

    out = out2.reshape(N, _OHP, Cout, _OWP)[:, :OH, :, :OW]
    return out.transpose(0, 2, 1, 3)


# speedup vs baseline: 1.0541x; 1.0541x over previous
import functools

import jax
import jax.numpy as jnp
from jax.experimental import pallas as pl
from jax.experimental.pallas import tpu as pltpu

_SQRT_2_OVER_PI = 0.7978845608028654
_GELU_C = 0.044715

_STRIDE = 3
_DIL = 2
_NPH = 3
_NTAP = 9
_UP = 56
_OWP = 32
_OHP = 32
_T_OF = (0, 2, 1)
_DT = (0, 1, 0)


def _gelu_tail(v1):
    v3 = v1 * v1
    v5 = (v3 * v1) * _GELU_C
    v7 = (v1 + v5) * _SQRT_2_OVER_PI
    v10 = (v1 * 0.5) * (1.0 + jnp.tanh(v7))
    return v10 + v5


def _conv_gelu_body(xs_ref, g_ref, b_ref, o_ref, *, lb):
    acc = jnp.zeros((_OHP, 128), jnp.float32)
    for p in range(_NTAP):
        for ph in range(_NPH):
            lhs = xs_ref[0, pl.ds(_DIL * p + _DT[ph], _OHP),
                         pl.ds(ph * lb, lb)]
            acc += jnp.dot(lhs, g_ref[_NPH * p + ph],
                           preferred_element_type=jnp.float32)
    o_ref[0] = _gelu_tail(acc + b_ref[...])


@jax.jit
def kernel(x, w1, b1):
    N, Cin, H, W = map(int, x.shape)
    Cout, _, KH, KW = map(int, w1.shape)
    OH = (H - _DIL * (KH - 1) - 1) // _STRIDE + 1
    OW = (W - _DIL * (KW - 1) - 1) // _STRIDE + 1
    LB = Cin * W

    xp = jnp.pad(x.astype(jnp.bfloat16),
                 ((0, 0), (0, 0), (0, _STRIDE * _UP - H), (0, 0)))
    xr = xp.reshape(N, Cin, _UP, _NPH, W)
    xs = xr.transpose(0, 2, 3, 1, 4).reshape(N, _UP, _NPH * LB)

    w8 = jnp.pad(w1.astype(jnp.bfloat16),
                 ((0, 0), (0, 0), (0, _NPH * _NTAP - KH), (0, 0)))
    w9 = w8.reshape(Cout, Cin, _NTAP, _NPH, KW)
    w10 = jnp.take(w9, jnp.array(_T_OF), axis=3)
    c_idx = jnp.arange(W)[:, None, None]
    w_idx = jnp.arange(_OWP)[None, :, None]
    k_idx = jnp.arange(KW)[None, None, :]
    sel = ((c_idx == _STRIDE * w_idx + _DIL * k_idx)
           & (w_idx < OW)).astype(jnp.bfloat16)
    g = jnp.einsum('oipTk,cwk->pTicow', w10, sel,
                   preferred_element_type=jnp.float32)
    g = g.reshape(_NPH * _NTAP, LB, Cout * _OWP).astype(jnp.bfloat16)

    bvec = jnp.repeat(b1.astype(jnp.float32), _OWP).reshape(1, Cout * _OWP)

    cost = pl.CostEstimate(
        flops=2 * N * _NTAP * _NPH * _OHP * LB * Cout * _OWP,
        transcendentals=N * _OHP * Cout * _OWP,
        bytes_accessed=xs.size * 2 + g.size * 2 + N * _OHP * Cout * _OWP * 4,
    )
    out2 = pl.pallas_call(
        functools.partial(_conv_gelu_body, lb=LB),
        out_shape=jax.ShapeDtypeStruct((N, _OHP, Cout * _OWP), jnp.float32),
        grid=(N,),
        in_specs=[
            pl.BlockSpec((1, _UP, _NPH * LB), lambda n: (n, 0, 0)),
            pl.BlockSpec((_NPH * _NTAP, LB, Cout * _OWP), lambda n: (0, 0, 0)),
            pl.BlockSpec((1, Cout * _OWP), lambda n: (0, 0)),
        ],
        out_specs=pl.BlockSpec((1, _OHP, Cout * _OWP), lambda n: (n, 0, 0)),
        compiler_params=pltpu.CompilerParams(
            dimension_semantics=("parallel",),
            vmem_limit_bytes=48 * 1024 * 1024),
        cost_estimate=cost,
    )(xs, g, bvec)

    out = out2.reshape(N, _OHP, Cout, _OWP)[:, :OH, :, :OW]
    return out.transpose(0, 2, 1, 3)
